# baseline (speedup 1.0000x reference)
import functools

import jax
import jax.numpy as jnp
from jax import lax
from jax.experimental import pallas as pl
from jax.experimental.pallas import tpu as pltpu
from jax.experimental.pallas import tpu_sc as plsc

BATCH = 16384
EMBED_DIM = 128
NUM_RELATIONS = 1000
REL_PAD = 1024

SC_ROWS = 8192

NUM_CORES = 2
NUM_SUBCORES = 16
NUM_WORKERS = NUM_CORES * NUM_SUBCORES
ROWS_PER_WORKER = SC_ROWS // NUM_WORKERS
CHUNK = 128
NUM_CHUNKS = ROWS_PER_WORKER // CHUNK
NBUF = 2
LANES = 16

TC_BLOCK = 256


def _sc_body(head_hbm, idx_hbm, tail_hbm, rel_hbm, out_hbm,
             idx_v, out_v, h_v, t_v, r_v, sems):
    wid = lax.axis_index("s") * NUM_CORES + lax.axis_index("c")
    base = wid * ROWS_PER_WORKER
    lane_iota = lax.iota(jnp.int32, LANES)

    pltpu.sync_copy(idx_hbm.at[pl.ds(base, ROWS_PER_WORKER)], idx_v)

    def fire(ci):
        b = ci % NBUF
        cbase = base + ci * CHUNK
        idx_slice = idx_v.at[pl.ds(ci * CHUNK, CHUNK)]
        return (
            pltpu.async_copy(rel_hbm.at[idx_slice], r_v.at[b], sems.at[b]),
            pltpu.async_copy(head_hbm.at[pl.ds(cbase, CHUNK)], h_v.at[b],
                             sems.at[b]),
            pltpu.async_copy(tail_hbm.at[pl.ds(cbase, CHUNK)], t_v.at[b],
                             sems.at[b]),
        )

    def compute(ci):
        b = ci % NBUF
        bvec = jnp.full((LANES,), b, jnp.int32)

        def group_body(g, carry2):
            rows = g * LANES + lane_iota

            def d_body(dd, carry3):
                acc, dvec = carry3
                hh = plsc.load_gather(h_v, [bvec, rows, dvec])
                rr = plsc.load_gather(r_v, [bvec, rows, dvec])
                tt = plsc.load_gather(t_v, [bvec, rows, dvec])
                return acc + (hh * rr) * tt, (dvec + 1) & (EMBED_DIM - 1)

            acc, _ = lax.fori_loop(
                0, EMBED_DIM, d_body,
                (jnp.zeros((LANES,), jnp.float32), lane_iota),
                unroll=8)
            out_v[pl.ds(ci * CHUNK + g * LANES, LANES)] = acc
            return carry2

        lax.fori_loop(0, CHUNK // LANES, group_body, 0)

    copies = fire(0)
    for ci in range(NUM_CHUNKS):
        nxt = fire(ci + 1) if ci + 1 < NUM_CHUNKS else ()
        for c in copies:
            c.wait()
        compute(ci)
        copies = nxt

    pltpu.sync_copy(out_v, out_hbm.at[pl.ds(base, ROWS_PER_WORKER)])


def _sc_call(head_e, rel_idx, tail_e, rel_embedding):
    mesh = plsc.VectorSubcoreMesh(core_axis_name="c", subcore_axis_name="s")
    kern = functools.partial(
        pl.kernel,
        mesh=mesh,
        compiler_params=pltpu.CompilerParams(needs_layout_passes=False),
        out_type=jax.ShapeDtypeStruct((SC_ROWS,), jnp.float32),
        scratch_types=[
            pltpu.VMEM((ROWS_PER_WORKER,), jnp.int32),
            pltpu.VMEM((ROWS_PER_WORKER,), jnp.float32),
            pltpu.VMEM((NBUF, CHUNK, EMBED_DIM), jnp.float32),
            pltpu.VMEM((NBUF, CHUNK, EMBED_DIM), jnp.float32),
            pltpu.VMEM((NBUF, CHUNK, EMBED_DIM), jnp.float32),
            pltpu.SemaphoreType.DMA((NBUF,)),
        ],
    )(_sc_body)
    return kern(head_e, rel_idx, tail_e, rel_embedding)


def _tc_body(h_ref, idx_ref, t_ref, tab_ref, out_ref):
    idx = idx_ref[...]
    onehot = (idx ==
              lax.broadcasted_iota(jnp.int32, (1, REL_PAD), 1)
              ).astype(jnp.float32)
    rel = jax.lax.dot_general(
        onehot, tab_ref[...],
        dimension_numbers=(((1,), (0,)), ((), ())),
        preferred_element_type=jnp.float32)
    prod = h_ref[...] * rel * t_ref[...]
    ones = jnp.ones((EMBED_DIM, 8), jnp.float32)
    red = jax.lax.dot_general(
        prod, ones,
        dimension_numbers=(((1,), (0,)), ((), ())),
        precision=jax.lax.Precision.DEFAULT,
        preferred_element_type=jnp.float32)
    out_ref[...] = red[:, :1]


def _tc_call(head_e, rel_idx_col, tail_e, rel_pad):
    off = SC_ROWS // TC_BLOCK
    grid = ((BATCH - SC_ROWS) // TC_BLOCK,)
    out = pl.pallas_call(
        _tc_body,
        grid=grid,
        in_specs=[
            pl.BlockSpec((TC_BLOCK, EMBED_DIM), lambda i: (off + i, 0)),
            pl.BlockSpec((TC_BLOCK, 1), lambda i: (off + i, 0)),
            pl.BlockSpec((TC_BLOCK, EMBED_DIM), lambda i: (off + i, 0)),
            pl.BlockSpec((REL_PAD, EMBED_DIM), lambda i: (0, 0)),
        ],
        out_specs=pl.BlockSpec((TC_BLOCK, 1), lambda i: (i, 0)),
        out_shape=jax.ShapeDtypeStruct((BATCH - SC_ROWS, 1), jnp.float32),
    )(head_e, rel_idx_col, tail_e, rel_pad)
    return out.reshape(-1)


@jax.jit
def _distmult(head_e, rel_idx, tail_e, rel_embedding):
    rel_pad = jnp.zeros((REL_PAD, EMBED_DIM), jnp.float32)
    rel_pad = rel_pad.at[:NUM_RELATIONS].set(rel_embedding)
    sc_out = _sc_call(head_e, rel_idx, tail_e, rel_embedding)
    tc_out = _tc_call(head_e, rel_idx.reshape(-1, 1), tail_e, rel_pad)
    return jnp.concatenate([sc_out, tc_out])


def kernel(head_e, rel_idx, tail_e, rel_embedding):
    return _distmult(head_e, rel_idx.astype(jnp.int32), tail_e,
                     rel_embedding)


# speedup vs baseline: 1.2383x; 1.2383x over previous
import functools

import jax
import jax.numpy as jnp
from jax import lax
from jax.experimental import pallas as pl
from jax.experimental.pallas import tpu as pltpu
from jax.experimental.pallas import tpu_sc as plsc

BATCH = 16384
EMBED_DIM = 128
NUM_RELATIONS = 1000
REL_PAD = 1024

SC_ROWS = 8192

NUM_CORES = 2
NUM_SUBCORES = 16
NUM_WORKERS = NUM_CORES * NUM_SUBCORES
ROWS_PER_WORKER = SC_ROWS // NUM_WORKERS
CHUNK = 128
NUM_CHUNKS = ROWS_PER_WORKER // CHUNK
NBUF = 2
LANES = 16

TC_BLOCK = 256


def _sc_body(head_hbm, idx_hbm, tail_hbm, rel_hbm, out_hbm,
             idx_v, out_v, h_v, t_v, r_v, sems):
    wid = lax.axis_index("s") * NUM_CORES + lax.axis_index("c")
    base = wid * ROWS_PER_WORKER
    lane_iota = lax.iota(jnp.int32, LANES)

    pltpu.sync_copy(idx_hbm.at[pl.ds(base, ROWS_PER_WORKER)], idx_v)

    def fire(ci):
        b = ci % NBUF
        cbase = base + ci * CHUNK
        idx_slice = idx_v.at[pl.ds(ci * CHUNK, CHUNK)]
        return (
            pltpu.async_copy(rel_hbm.at[idx_slice], r_v.at[b], sems.at[b]),
            pltpu.async_copy(head_hbm.at[pl.ds(cbase, CHUNK)], h_v.at[b],
                             sems.at[b]),
            pltpu.async_copy(tail_hbm.at[pl.ds(cbase, CHUNK)], t_v.at[b],
                             sems.at[b]),
        )

    def compute(ci):
        b = ci % NBUF
        bvec = jnp.full((LANES,), b, jnp.int32)

        def group_body(g, carry2):
            rows = g * LANES + lane_iota

            def d_body(dd, carry3):
                acc, dvec = carry3
                hh = plsc.load_gather(h_v, [bvec, rows, dvec])
                rr = plsc.load_gather(r_v, [bvec, rows, dvec])
                tt = plsc.load_gather(t_v, [bvec, rows, dvec])
                return acc + (hh * rr) * tt, (dvec + 1) & (EMBED_DIM - 1)

            acc, _ = lax.fori_loop(
                0, EMBED_DIM, d_body,
                (jnp.zeros((LANES,), jnp.float32), lane_iota),
                unroll=8)
            out_v[pl.ds(ci * CHUNK + g * LANES, LANES)] = acc
            return carry2

        lax.fori_loop(0, CHUNK // LANES, group_body, 0)

    copies = fire(0)
    for ci in range(NUM_CHUNKS):
        nxt = fire(ci + 1) if ci + 1 < NUM_CHUNKS else ()
        for c in copies:
            c.wait()
        compute(ci)
        copies = nxt

    pltpu.sync_copy(out_v, out_hbm.at[pl.ds(base, ROWS_PER_WORKER)])


def _sc_call(head_e, rel_idx, tail_e, rel_embedding):
    mesh = plsc.VectorSubcoreMesh(core_axis_name="c", subcore_axis_name="s")
    kern = functools.partial(
        pl.kernel,
        mesh=mesh,
        compiler_params=pltpu.CompilerParams(needs_layout_passes=False),
        out_type=jax.ShapeDtypeStruct((SC_ROWS,), jnp.float32),
        scratch_types=[
            pltpu.VMEM((ROWS_PER_WORKER,), jnp.int32),
            pltpu.VMEM((ROWS_PER_WORKER,), jnp.float32),
            pltpu.VMEM((NBUF, CHUNK, EMBED_DIM), jnp.float32),
            pltpu.VMEM((NBUF, CHUNK, EMBED_DIM), jnp.float32),
            pltpu.VMEM((NBUF, CHUNK, EMBED_DIM), jnp.float32),
            pltpu.SemaphoreType.DMA((NBUF,)),
        ],
    )(_sc_body)
    return kern(head_e, rel_idx, tail_e, rel_embedding)


def _tc_body(h_ref, idx_ref, t_ref, tab_hbm, out_ref, tab_v, sem):
    @pl.when(pl.program_id(0) == 0)
    def _():
        cp = pltpu.make_async_copy(tab_hbm,
                                   tab_v.at[pl.ds(0, NUM_RELATIONS)], sem)
        cp.start()
        tab_v[pl.ds(NUM_RELATIONS, REL_PAD - NUM_RELATIONS), :] = jnp.zeros(
            (REL_PAD - NUM_RELATIONS, EMBED_DIM), jnp.float32)
        cp.wait()

    idx = idx_ref[...]
    onehot = (idx[:, None] ==
              lax.broadcasted_iota(jnp.int32, (1, REL_PAD), 1)
              ).astype(jnp.float32)
    rel = jax.lax.dot_general(
        onehot, tab_v[...],
        dimension_numbers=(((1,), (0,)), ((), ())),
        preferred_element_type=jnp.float32)
    out_ref[...] = jnp.sum(h_ref[...] * rel * t_ref[...], axis=1)


def _tc_call(head_e, rel_idx, tail_e, rel_embedding):
    off = SC_ROWS // TC_BLOCK
    grid = ((BATCH - SC_ROWS) // TC_BLOCK,)
    return pl.pallas_call(
        _tc_body,
        grid=grid,
        in_specs=[
            pl.BlockSpec((TC_BLOCK, EMBED_DIM), lambda i: (off + i, 0)),
            pl.BlockSpec((TC_BLOCK,), lambda i: (off + i,)),
            pl.BlockSpec((TC_BLOCK, EMBED_DIM), lambda i: (off + i, 0)),
            pl.BlockSpec(memory_space=pl.ANY),
        ],
        out_specs=pl.BlockSpec((TC_BLOCK,), lambda i: (i,)),
        out_shape=jax.ShapeDtypeStruct((BATCH - SC_ROWS,), jnp.float32),
        scratch_shapes=[
            pltpu.VMEM((REL_PAD, EMBED_DIM), jnp.float32),
            pltpu.SemaphoreType.DMA,
        ],
    )(head_e, rel_idx, tail_e, rel_embedding)


@jax.jit
def _distmult(head_e, rel_idx, tail_e, rel_embedding):
    sc_out = _sc_call(head_e, rel_idx, tail_e, rel_embedding)
    tc_out = _tc_call(head_e, rel_idx, tail_e, rel_embedding)
    return jnp.concatenate([sc_out, tc_out])


def kernel(head_e, rel_idx, tail_e, rel_embedding):
    return _distmult(head_e, rel_idx.astype(jnp.int32), tail_e,
                     rel_embedding)


# speedup vs baseline: 1.6335x; 1.3191x over previous
import functools

import jax
import jax.numpy as jnp
from jax import lax
from jax.experimental import pallas as pl
from jax.experimental.pallas import tpu as pltpu
from jax.experimental.pallas import tpu_sc as plsc

BATCH = 16384
EMBED_DIM = 128
NUM_RELATIONS = 1000
REL_PAD = 1024

SC_ROWS = 12288

NUM_CORES = 2
NUM_SUBCORES = 16
NUM_WORKERS = NUM_CORES * NUM_SUBCORES
ROWS_PER_WORKER = SC_ROWS // NUM_WORKERS
CHUNK = 128
NUM_CHUNKS = ROWS_PER_WORKER // CHUNK
NBUF = 2
LANES = 16

TC_BLOCK = 256


def _sc_body(head_hbm, idx_hbm, tail_hbm, rel_hbm, out_hbm,
             idx_v, out_v, h_v, t_v, r_v, sems):
    wid = lax.axis_index("s") * NUM_CORES + lax.axis_index("c")
    base = wid * ROWS_PER_WORKER
    lane_iota = lax.iota(jnp.int32, LANES)

    pltpu.sync_copy(idx_hbm.at[pl.ds(base, ROWS_PER_WORKER)], idx_v)

    def fire(ci):
        b = ci % NBUF
        cbase = base + ci * CHUNK
        idx_slice = idx_v.at[pl.ds(ci * CHUNK, CHUNK)]
        return (
            pltpu.async_copy(rel_hbm.at[idx_slice], r_v.at[b], sems.at[b]),
            pltpu.async_copy(head_hbm.at[pl.ds(cbase, CHUNK)], h_v.at[b],
                             sems.at[b]),
            pltpu.async_copy(tail_hbm.at[pl.ds(cbase, CHUNK)], t_v.at[b],
                             sems.at[b]),
        )

    def compute(ci):
        b = ci % NBUF
        bvec = jnp.full((LANES,), b, jnp.int32)

        def group_body(g, carry2):
            rows = g * LANES + lane_iota

            def d_body(dd, carry3):
                acc, dvec = carry3
                hh = plsc.load_gather(h_v, [bvec, rows, dvec])
                rr = plsc.load_gather(r_v, [bvec, rows, dvec])
                tt = plsc.load_gather(t_v, [bvec, rows, dvec])
                return acc + (hh * rr) * tt, (dvec + 1) & (EMBED_DIM - 1)

            acc, _ = lax.fori_loop(
                0, EMBED_DIM, d_body,
                (jnp.zeros((LANES,), jnp.float32), lane_iota),
                unroll=8)
            out_v[pl.ds(ci * CHUNK + g * LANES, LANES)] = acc
            return carry2

        lax.fori_loop(0, CHUNK // LANES, group_body, 0)

    copies = fire(0)
    for ci in range(NUM_CHUNKS):
        nxt = fire(ci + 1) if ci + 1 < NUM_CHUNKS else ()
        for c in copies:
            c.wait()
        compute(ci)
        copies = nxt

    pltpu.sync_copy(out_v, out_hbm.at[pl.ds(base, ROWS_PER_WORKER)])


def _sc_call(head_e, rel_idx, tail_e, rel_embedding):
    mesh = plsc.VectorSubcoreMesh(core_axis_name="c", subcore_axis_name="s")
    kern = functools.partial(
        pl.kernel,
        mesh=mesh,
        compiler_params=pltpu.CompilerParams(needs_layout_passes=False),
        out_type=jax.ShapeDtypeStruct((SC_ROWS,), jnp.float32),
        scratch_types=[
            pltpu.VMEM((ROWS_PER_WORKER,), jnp.int32),
            pltpu.VMEM((ROWS_PER_WORKER,), jnp.float32),
            pltpu.VMEM((NBUF, CHUNK, EMBED_DIM), jnp.float32),
            pltpu.VMEM((NBUF, CHUNK, EMBED_DIM), jnp.float32),
            pltpu.VMEM((NBUF, CHUNK, EMBED_DIM), jnp.float32),
            pltpu.SemaphoreType.DMA((NBUF,)),
        ],
    )(_sc_body)
    return kern(head_e, rel_idx, tail_e, rel_embedding)


def _tc_body(h_ref, idx_ref, t_ref, tab_hbm, out_ref, tab_v, sem):
    @pl.when(pl.program_id(0) == 0)
    def _():
        cp = pltpu.make_async_copy(tab_hbm,
                                   tab_v.at[pl.ds(0, NUM_RELATIONS)], sem)
        cp.start()
        tab_v[pl.ds(NUM_RELATIONS, REL_PAD - NUM_RELATIONS), :] = jnp.zeros(
            (REL_PAD - NUM_RELATIONS, EMBED_DIM), jnp.float32)
        cp.wait()

    idx = idx_ref[...]
    onehot = (idx[:, None] ==
              lax.broadcasted_iota(jnp.int32, (1, REL_PAD), 1)
              ).astype(jnp.float32)
    rel = jax.lax.dot_general(
        onehot, tab_v[...],
        dimension_numbers=(((1,), (0,)), ((), ())),
        preferred_element_type=jnp.float32)
    out_ref[...] = jnp.sum(h_ref[...] * rel * t_ref[...], axis=1)


def _tc_call(head_e, rel_idx, tail_e, rel_embedding):
    off = SC_ROWS // TC_BLOCK
    grid = ((BATCH - SC_ROWS) // TC_BLOCK,)
    return pl.pallas_call(
        _tc_body,
        grid=grid,
        in_specs=[
            pl.BlockSpec((TC_BLOCK, EMBED_DIM), lambda i: (off + i, 0)),
            pl.BlockSpec((TC_BLOCK,), lambda i: (off + i,)),
            pl.BlockSpec((TC_BLOCK, EMBED_DIM), lambda i: (off + i, 0)),
            pl.BlockSpec(memory_space=pl.ANY),
        ],
        out_specs=pl.BlockSpec((TC_BLOCK,), lambda i: (i,)),
        out_shape=jax.ShapeDtypeStruct((BATCH - SC_ROWS,), jnp.float32),
        scratch_shapes=[
            pltpu.VMEM((REL_PAD, EMBED_DIM), jnp.float32),
            pltpu.SemaphoreType.DMA,
        ],
    )(head_e, rel_idx, tail_e, rel_embedding)


@jax.jit
def _distmult(head_e, rel_idx, tail_e, rel_embedding):
    sc_out = _sc_call(head_e, rel_idx, tail_e, rel_embedding)
    tc_out = _tc_call(head_e, rel_idx, tail_e, rel_embedding)
    return jnp.concatenate([sc_out, tc_out])


def kernel(head_e, rel_idx, tail_e, rel_embedding):
    return _distmult(head_e, rel_idx.astype(jnp.int32), tail_e,
                     rel_embedding)


# speedup vs baseline: 1.6550x; 1.0132x over previous
import functools

import jax
import jax.numpy as jnp
from jax import lax
from jax.experimental import pallas as pl
from jax.experimental.pallas import tpu as pltpu
from jax.experimental.pallas import tpu_sc as plsc

BATCH = 16384
EMBED_DIM = 128
NUM_RELATIONS = 1000
REL_PAD = 1024

SC_ROWS = 12288

NUM_CORES = 2
NUM_SUBCORES = 16
NUM_WORKERS = NUM_CORES * NUM_SUBCORES
ROWS_PER_WORKER = SC_ROWS // NUM_WORKERS
CHUNK = 128
NUM_CHUNKS = ROWS_PER_WORKER // CHUNK
NBUF = 2
LANES = 16

TC_BLOCK = 512


def _sc_body(head_hbm, idx_hbm, tail_hbm, rel_hbm, out_hbm,
             idx_v, out_v, h_v, t_v, r_v, sems):
    wid = lax.axis_index("s") * NUM_CORES + lax.axis_index("c")
    base = wid * ROWS_PER_WORKER
    lane_iota = lax.iota(jnp.int32, LANES)

    pltpu.sync_copy(idx_hbm.at[pl.ds(base, ROWS_PER_WORKER)], idx_v)

    def fire(ci):
        b = ci % NBUF
        cbase = base + ci * CHUNK
        idx_slice = idx_v.at[pl.ds(ci * CHUNK, CHUNK)]
        return (
            pltpu.async_copy(rel_hbm.at[idx_slice], r_v.at[b], sems.at[b]),
            pltpu.async_copy(head_hbm.at[pl.ds(cbase, CHUNK)], h_v.at[b],
                             sems.at[b]),
            pltpu.async_copy(tail_hbm.at[pl.ds(cbase, CHUNK)], t_v.at[b],
                             sems.at[b]),
        )

    def compute(ci):
        b = ci % NBUF
        bvec = jnp.full((LANES,), b, jnp.int32)

        def group_body(g, carry2):
            rows = g * LANES + lane_iota

            def d_body(dd, carry3):
                acc, dvec = carry3
                hh = plsc.load_gather(h_v, [bvec, rows, dvec])
                rr = plsc.load_gather(r_v, [bvec, rows, dvec])
                tt = plsc.load_gather(t_v, [bvec, rows, dvec])
                return acc + (hh * rr) * tt, (dvec + 1) & (EMBED_DIM - 1)

            acc, _ = lax.fori_loop(
                0, EMBED_DIM, d_body,
                (jnp.zeros((LANES,), jnp.float32), lane_iota),
                unroll=8)
            out_v[pl.ds(ci * CHUNK + g * LANES, LANES)] = acc
            return carry2

        lax.fori_loop(0, CHUNK // LANES, group_body, 0)

    copies = fire(0)
    for ci in range(NUM_CHUNKS):
        nxt = fire(ci + 1) if ci + 1 < NUM_CHUNKS else ()
        for c in copies:
            c.wait()
        compute(ci)
        copies = nxt

    pltpu.sync_copy(out_v, out_hbm.at[pl.ds(base, ROWS_PER_WORKER)])


def _sc_call(head_e, rel_idx, tail_e, rel_embedding):
    mesh = plsc.VectorSubcoreMesh(core_axis_name="c", subcore_axis_name="s")
    kern = functools.partial(
        pl.kernel,
        mesh=mesh,
        compiler_params=pltpu.CompilerParams(needs_layout_passes=False),
        out_type=jax.ShapeDtypeStruct((SC_ROWS,), jnp.float32),
        scratch_types=[
            pltpu.VMEM((ROWS_PER_WORKER,), jnp.int32),
            pltpu.VMEM((ROWS_PER_WORKER,), jnp.float32),
            pltpu.VMEM((NBUF, CHUNK, EMBED_DIM), jnp.float32),
            pltpu.VMEM((NBUF, CHUNK, EMBED_DIM), jnp.float32),
            pltpu.VMEM((NBUF, CHUNK, EMBED_DIM), jnp.float32),
            pltpu.SemaphoreType.DMA((NBUF,)),
        ],
    )(_sc_body)
    return kern(head_e, rel_idx, tail_e, rel_embedding)


def _tc_body(h_ref, idx_ref, t_ref, tab_hbm, out_ref, tab_v, sem):
    @pl.when(pl.program_id(0) == 0)
    def _():
        cp = pltpu.make_async_copy(tab_hbm,
                                   tab_v.at[pl.ds(0, NUM_RELATIONS)], sem)
        cp.start()
        tab_v[pl.ds(NUM_RELATIONS, REL_PAD - NUM_RELATIONS), :] = jnp.zeros(
            (REL_PAD - NUM_RELATIONS, EMBED_DIM), jnp.float32)
        cp.wait()

    idx = idx_ref[...]
    onehot = (idx[:, None] ==
              lax.broadcasted_iota(jnp.int32, (1, REL_PAD), 1)
              ).astype(jnp.float32)
    rel = jax.lax.dot_general(
        onehot, tab_v[...],
        dimension_numbers=(((1,), (0,)), ((), ())),
        preferred_element_type=jnp.float32)
    out_ref[...] = jnp.sum(h_ref[...] * rel * t_ref[...], axis=1)


def _tc_call(head_e, rel_idx, tail_e, rel_embedding):
    off = SC_ROWS // TC_BLOCK
    grid = ((BATCH - SC_ROWS) // TC_BLOCK,)
    return pl.pallas_call(
        _tc_body,
        grid=grid,
        in_specs=[
            pl.BlockSpec((TC_BLOCK, EMBED_DIM), lambda i: (off + i, 0)),
            pl.BlockSpec((TC_BLOCK,), lambda i: (off + i,)),
            pl.BlockSpec((TC_BLOCK, EMBED_DIM), lambda i: (off + i, 0)),
            pl.BlockSpec(memory_space=pl.ANY),
        ],
        out_specs=pl.BlockSpec((TC_BLOCK,), lambda i: (i,)),
        out_shape=jax.ShapeDtypeStruct((BATCH - SC_ROWS,), jnp.float32),
        scratch_shapes=[
            pltpu.VMEM((REL_PAD, EMBED_DIM), jnp.float32),
            pltpu.SemaphoreType.DMA,
        ],
    )(head_e, rel_idx, tail_e, rel_embedding)


@jax.jit
def _distmult(head_e, rel_idx, tail_e, rel_embedding):
    sc_out = _sc_call(head_e, rel_idx, tail_e, rel_embedding)
    tc_out = _tc_call(head_e, rel_idx, tail_e, rel_embedding)
    return jnp.concatenate([sc_out, tc_out])


def kernel(head_e, rel_idx, tail_e, rel_embedding):
    return _distmult(head_e, rel_idx.astype(jnp.int32), tail_e,
                     rel_embedding)


# speedup vs baseline: 1.6712x; 1.0098x over previous
import functools

import jax
import jax.numpy as jnp
from jax import lax
from jax.experimental import pallas as pl
from jax.experimental.pallas import tpu as pltpu
from jax.experimental.pallas import tpu_sc as plsc

BATCH = 16384
EMBED_DIM = 128
NUM_RELATIONS = 1000
REL_PAD = 1024

SC_ROWS = 12288

NUM_CORES = 2
NUM_SUBCORES = 16
NUM_WORKERS = NUM_CORES * NUM_SUBCORES
ROWS_PER_WORKER = SC_ROWS // NUM_WORKERS
CHUNK = 64
NUM_CHUNKS = ROWS_PER_WORKER // CHUNK
NBUF = 3
LANES = 16

TC_BLOCK = 512


def _sc_body(head_hbm, idx_hbm, tail_hbm, rel_hbm, out_hbm,
             idx_v, out_v, h_v, t_v, r_v, sems):
    wid = lax.axis_index("s") * NUM_CORES + lax.axis_index("c")
    base = wid * ROWS_PER_WORKER
    lane_iota = lax.iota(jnp.int32, LANES)

    pltpu.sync_copy(idx_hbm.at[pl.ds(base, ROWS_PER_WORKER)], idx_v)

    def fire(ci):
        b = ci % NBUF
        cbase = base + ci * CHUNK
        idx_slice = idx_v.at[pl.ds(ci * CHUNK, CHUNK)]
        return (
            pltpu.async_copy(rel_hbm.at[idx_slice], r_v.at[b], sems.at[b]),
            pltpu.async_copy(head_hbm.at[pl.ds(cbase, CHUNK)], h_v.at[b],
                             sems.at[b]),
            pltpu.async_copy(tail_hbm.at[pl.ds(cbase, CHUNK)], t_v.at[b],
                             sems.at[b]),
        )

    def compute(ci):
        b = ci % NBUF
        bvec = jnp.full((LANES,), b, jnp.int32)

        def group_body(g, carry2):
            rows = g * LANES + lane_iota

            def d_body(dd, carry3):
                acc, dvec = carry3
                hh = plsc.load_gather(h_v, [bvec, rows, dvec])
                rr = plsc.load_gather(r_v, [bvec, rows, dvec])
                tt = plsc.load_gather(t_v, [bvec, rows, dvec])
                return acc + (hh * rr) * tt, (dvec + 1) & (EMBED_DIM - 1)

            acc, _ = lax.fori_loop(
                0, EMBED_DIM, d_body,
                (jnp.zeros((LANES,), jnp.float32), lane_iota),
                unroll=8)
            out_v[pl.ds(ci * CHUNK + g * LANES, LANES)] = acc
            return carry2

        lax.fori_loop(0, CHUNK // LANES, group_body, 0)

    copies = fire(0)
    for ci in range(NUM_CHUNKS):
        nxt = fire(ci + 1) if ci + 1 < NUM_CHUNKS else ()
        for c in copies:
            c.wait()
        compute(ci)
        copies = nxt

    pltpu.sync_copy(out_v, out_hbm.at[pl.ds(base, ROWS_PER_WORKER)])


def _sc_call(head_e, rel_idx, tail_e, rel_embedding):
    mesh = plsc.VectorSubcoreMesh(core_axis_name="c", subcore_axis_name="s")
    kern = functools.partial(
        pl.kernel,
        mesh=mesh,
        compiler_params=pltpu.CompilerParams(needs_layout_passes=False),
        out_type=jax.ShapeDtypeStruct((SC_ROWS,), jnp.float32),
        scratch_types=[
            pltpu.VMEM((ROWS_PER_WORKER,), jnp.int32),
            pltpu.VMEM((ROWS_PER_WORKER,), jnp.float32),
            pltpu.VMEM((NBUF, CHUNK, EMBED_DIM), jnp.float32),
            pltpu.VMEM((NBUF, CHUNK, EMBED_DIM), jnp.float32),
            pltpu.VMEM((NBUF, CHUNK, EMBED_DIM), jnp.float32),
            pltpu.SemaphoreType.DMA((NBUF,)),
        ],
    )(_sc_body)
    return kern(head_e, rel_idx, tail_e, rel_embedding)


def _tc_body(h_ref, idx_ref, t_ref, tab_hbm, out_ref, tab_v, sem):
    @pl.when(pl.program_id(0) == 0)
    def _():
        cp = pltpu.make_async_copy(tab_hbm,
                                   tab_v.at[pl.ds(0, NUM_RELATIONS)], sem)
        cp.start()
        tab_v[pl.ds(NUM_RELATIONS, REL_PAD - NUM_RELATIONS), :] = jnp.zeros(
            (REL_PAD - NUM_RELATIONS, EMBED_DIM), jnp.float32)
        cp.wait()

    idx = idx_ref[...]
    onehot = (idx[:, None] ==
              lax.broadcasted_iota(jnp.int32, (1, REL_PAD), 1)
              ).astype(jnp.float32)
    rel = jax.lax.dot_general(
        onehot, tab_v[...],
        dimension_numbers=(((1,), (0,)), ((), ())),
        preferred_element_type=jnp.float32)
    out_ref[...] = jnp.sum(h_ref[...] * rel * t_ref[...], axis=1)


def _tc_call(head_e, rel_idx, tail_e, rel_embedding):
    off = SC_ROWS // TC_BLOCK
    grid = ((BATCH - SC_ROWS) // TC_BLOCK,)
    return pl.pallas_call(
        _tc_body,
        grid=grid,
        in_specs=[
            pl.BlockSpec((TC_BLOCK, EMBED_DIM), lambda i: (off + i, 0)),
            pl.BlockSpec((TC_BLOCK,), lambda i: (off + i,)),
            pl.BlockSpec((TC_BLOCK, EMBED_DIM), lambda i: (off + i, 0)),
            pl.BlockSpec(memory_space=pl.ANY),
        ],
        out_specs=pl.BlockSpec((TC_BLOCK,), lambda i: (i,)),
        out_shape=jax.ShapeDtypeStruct((BATCH - SC_ROWS,), jnp.float32),
        scratch_shapes=[
            pltpu.VMEM((REL_PAD, EMBED_DIM), jnp.float32),
            pltpu.SemaphoreType.DMA,
        ],
    )(head_e, rel_idx, tail_e, rel_embedding)


@jax.jit
def _distmult(head_e, rel_idx, tail_e, rel_embedding):
    sc_out = _sc_call(head_e, rel_idx, tail_e, rel_embedding)
    tc_out = _tc_call(head_e, rel_idx, tail_e, rel_embedding)
    return jnp.concatenate([sc_out, tc_out])


def kernel(head_e, rel_idx, tail_e, rel_embedding):
    return _distmult(head_e, rel_idx.astype(jnp.int32), tail_e,
                     rel_embedding)
